# Initial kernel scaffold; baseline (speedup 1.0000x reference)
import jax
import jax.numpy as jnp
from jax.experimental import pallas as pl


def kernel(x_embed):
    raise NotImplementedError("write your pallas kernel here")


# speedup vs baseline: 18.5955x; 18.5955x over previous
import functools

import jax
import jax.numpy as jnp
import numpy as np
from jax.experimental import pallas as pl
from jax.experimental.pallas import tpu as pltpu

_NF = 100
_D = 128
_ROW_NP, _COL_NP = np.triu_indices(_NF, k=1)
_P = _ROW_NP.size
_FLAT_IDX = (_ROW_NP * _NF + _COL_NP).astype(np.int32)

_BBLK = 16


def _gram_body(x_ref, g_ref):
    for b in range(_BBLK):
        xb = x_ref[b]
        g_ref[b] = jax.lax.dot_general(
            xb, xb, (((1,), (1,)), ((), ())),
            preferred_element_type=jnp.float32)


def _gram(x_embed):
    B = x_embed.shape[0]
    return pl.pallas_call(
        _gram_body,
        grid=(B // _BBLK,),
        in_specs=[pl.BlockSpec((_BBLK, _NF, _D), lambda i: (i, 0, 0))],
        out_specs=pl.BlockSpec((_BBLK, _NF, _NF), lambda i: (i, 0, 0)),
        out_shape=jax.ShapeDtypeStruct((B, _NF, _NF), jnp.float32),
    )(x_embed)


def kernel(x_embed):
    B = x_embed.shape[0]
    g = _gram(x_embed).reshape(B, _NF * _NF)
    return jnp.take(g, jnp.asarray(_FLAT_IDX), axis=1)
